# baseline (speedup 1.0000x reference)
import jax
import jax.numpy as jnp
from jax.experimental import pallas as pl

_ROWS = 16384
_COLS = 2048
_BLOCK_ROWS = 1024


def _masked_tanh_kernel(x_ref, m_ref, o_ref):
    x = x_ref[...]
    m = m_ref[...]
    o_ref[...] = jnp.where(m != 0.0, jnp.tanh(x), x)


def kernel(x, mask):
    m = mask.astype(jnp.float32).reshape(1, _COLS)
    return pl.pallas_call(
        _masked_tanh_kernel,
        grid=(_ROWS // _BLOCK_ROWS,),
        in_specs=[
            pl.BlockSpec((_BLOCK_ROWS, _COLS), lambda i: (i, 0)),
            pl.BlockSpec((1, _COLS), lambda i: (0, 0)),
        ],
        out_specs=pl.BlockSpec((_BLOCK_ROWS, _COLS), lambda i: (i, 0)),
        out_shape=jax.ShapeDtypeStruct((_ROWS, _COLS), jnp.float32),
    )(x, m)
